# Initial kernel scaffold; baseline (speedup 1.0000x reference)
import jax
import jax.numpy as jnp
from jax.experimental import pallas as pl


def kernel(logits, emb_weight):
    raise NotImplementedError("write your pallas kernel here")


# speedup vs baseline: 2.8672x; 2.8672x over previous
import functools

import jax
import jax.numpy as jnp
from jax import lax
from jax.experimental import pallas as pl
from jax.experimental.pallas import tpu as pltpu
from jax.experimental.pallas import tpu_sc as plsc

K = 50
KPAD = 56
NSEG = 1024
SEGW = 125
D = 2048
B = 64
NC = 2
NS = 16
ROWS_PER_W = B // (NC * NS)


def _topk_body(x_ref, w_ref, i_ref):
    x = x_ref[0]
    segmax = jnp.max(x, axis=1)
    seg_iota = lax.broadcasted_iota(jnp.int32, (NSEG,), 0)
    lane_iota = lax.broadcasted_iota(jnp.int32, (1, SEGW), 1)
    pad_iota = lax.broadcasted_iota(jnp.int32, (KPAD,), 0)
    big = jnp.int32(2 ** 30)
    neg = jnp.float32(-jnp.inf)

    vals = jnp.full((KPAD,), neg, jnp.float32)
    idxs = jnp.zeros((KPAD,), jnp.int32)
    gm0 = jnp.float32(0.0)
    for i in range(K):
        gm = jnp.max(segmax)
        if i == 0:
            gm0 = gm
        sid = jnp.min(jnp.where(segmax == gm, seg_iota, big))
        row = x_ref[0, pl.ds(sid, 1), :]
        pos = jnp.min(jnp.where(row == gm, lane_iota, big))
        gidx = sid * SEGW + pos
        vals = jnp.where(pad_iota == i, gm, vals)
        idxs = jnp.where(pad_iota == i, gidx, idxs)
        masked = jnp.where(lane_iota == pos, neg, row)
        x_ref[0, pl.ds(sid, 1), :] = masked
        segmax = jnp.where(seg_iota == sid, jnp.max(masked), segmax)

    w = jnp.exp(vals - gm0)
    w = w / jnp.sum(w)
    w_ref[0, :, :] = jnp.broadcast_to(w[:, None], (KPAD, 16))
    i_ref[0, 0, :] = idxs


def _topk(logits):
    x3 = logits.reshape(B, NSEG, SEGW)
    w, idx = pl.pallas_call(
        _topk_body,
        grid=(B,),
        in_specs=[pl.BlockSpec((1, NSEG, SEGW), lambda i: (i, 0, 0))],
        out_specs=[
            pl.BlockSpec((1, KPAD, 16), lambda i: (i, 0, 0)),
            pl.BlockSpec((1, 1, KPAD), lambda i: (i, 0, 0)),
        ],
        out_shape=[
            jax.ShapeDtypeStruct((B, KPAD, 16), jnp.float32),
            jax.ShapeDtypeStruct((B, 1, KPAD), jnp.int32),
        ],
        compiler_params=pltpu.CompilerParams(
            dimension_semantics=("arbitrary",)),
    )(x3)
    return w.reshape(B * KPAD * 16), idx.reshape(B * KPAD)


def _mix_body(emb_hbm, idx_hbm, w_hbm, out_hbm, idx_v, w_v, rows_v, acc_v, sem):
    wid = lax.axis_index("s") * NC + lax.axis_index("c")
    for r in range(ROWS_PER_W):
        row = wid * ROWS_PER_W + r
        pltpu.sync_copy(idx_hbm.at[pl.ds(row * KPAD, KPAD)], idx_v)
        pltpu.sync_copy(w_hbm.at[pl.ds(row * KPAD * 16, KPAD * 16)], w_v)
        pltpu.async_copy(emb_hbm.at[idx_v], rows_v, sem).wait()
        for c in range(D // 16):
            acc_v[pl.ds(c * 16, 16)] = jnp.zeros((16,), jnp.float32)

        def body(j, carry):
            wb = w_v[pl.ds(j * 16, 16)]
            for c in range(D // 16):
                sl = pl.ds(c * 16, 16)
                acc_v[sl] = acc_v[sl] + wb * rows_v[j, sl]
            return carry

        lax.fori_loop(0, KPAD, body, 0)
        pltpu.sync_copy(acc_v, out_hbm.at[row])


_MIX_CACHE = []


def _mix(emb_weight, idx, w):
    if not _MIX_CACHE:
        _MIX_CACHE.append(functools.partial(
            pl.kernel,
            mesh=plsc.VectorSubcoreMesh(core_axis_name="c", subcore_axis_name="s"),
            out_type=jax.ShapeDtypeStruct((B, D), jnp.float32),
            scratch_types=[
                pltpu.VMEM((KPAD,), jnp.int32),
                pltpu.VMEM((KPAD * 16,), jnp.float32),
                pltpu.VMEM((KPAD, D), jnp.float32),
                pltpu.VMEM((D,), jnp.float32),
                pltpu.SemaphoreType.DMA,
            ],
        )(_mix_body))
    return _MIX_CACHE[0](emb_weight, idx, w)


def kernel(logits, emb_weight):
    assert logits.shape == (B, 128000) and emb_weight.shape == (128000, D)
    w, idx = _topk(logits)
    return _mix(emb_weight, idx, w)


# speedup vs baseline: 3.4518x; 1.2039x over previous
import functools

import jax
import jax.numpy as jnp
from jax import lax
from jax.experimental import pallas as pl
from jax.experimental.pallas import tpu as pltpu
from jax.experimental.pallas import tpu_sc as plsc

K = 50
KPAD = 56
NSEG = 1024
SEGW = 125
D = 2048
B = 64
NC = 2
NS = 16
ROWS_PER_W = B // (NC * NS)


RB = 8


def _topk_body(x_ref, w_ref, i_ref):
    seg_iota = lax.broadcasted_iota(jnp.int32, (NSEG,), 0)
    lane_iota = lax.broadcasted_iota(jnp.int32, (1, SEGW), 1)
    pad_iota = lax.broadcasted_iota(jnp.int32, (KPAD,), 0)
    big = jnp.int32(2 ** 30)
    neg = jnp.float32(-jnp.inf)

    segmax = [jnp.max(x_ref[r], axis=1) for r in range(RB)]
    vals = [jnp.full((KPAD,), neg, jnp.float32) for _ in range(RB)]
    idxs = [jnp.zeros((KPAD,), jnp.int32) for _ in range(RB)]
    gm0 = [None] * RB
    for i in range(K):
        for r in range(RB):
            gm = jnp.max(segmax[r])
            if i == 0:
                gm0[r] = gm
            sid = jnp.min(jnp.where(segmax[r] == gm, seg_iota, big))
            row = x_ref[r, pl.ds(sid, 1), :]
            pos = jnp.min(jnp.where(row == gm, lane_iota, big))
            vals[r] = jnp.where(pad_iota == i, gm, vals[r])
            idxs[r] = jnp.where(pad_iota == i, sid * SEGW + pos, idxs[r])
            masked = jnp.where(lane_iota == pos, neg, row)
            x_ref[r, pl.ds(sid, 1), :] = masked
            segmax[r] = jnp.where(seg_iota == sid, jnp.max(masked), segmax[r])

    for r in range(RB):
        w = jnp.exp(vals[r] - gm0[r])
        w = w / jnp.sum(w)
        w_ref[r, :, :] = jnp.broadcast_to(w[:, None], (KPAD, 16))
        i_ref[r, 0, :] = idxs[r]


def _topk(logits):
    x3 = logits.reshape(B, NSEG, SEGW)
    w, idx = pl.pallas_call(
        _topk_body,
        grid=(B // RB,),
        in_specs=[pl.BlockSpec((RB, NSEG, SEGW), lambda i: (i, 0, 0))],
        out_specs=[
            pl.BlockSpec((RB, KPAD, 16), lambda i: (i, 0, 0)),
            pl.BlockSpec((RB, 1, KPAD), lambda i: (i, 0, 0)),
        ],
        out_shape=[
            jax.ShapeDtypeStruct((B, KPAD, 16), jnp.float32),
            jax.ShapeDtypeStruct((B, 1, KPAD), jnp.int32),
        ],
        compiler_params=pltpu.CompilerParams(
            dimension_semantics=("parallel",)),
    )(x3)
    return w.reshape(B * KPAD * 16), idx.reshape(B * KPAD)


def _mix_body(emb_hbm, idx_hbm, w_hbm, out_hbm, idx_v, w_v, rows_v, acc_v, sem):
    wid = lax.axis_index("s") * NC + lax.axis_index("c")
    for r in range(ROWS_PER_W):
        row = wid * ROWS_PER_W + r
        pltpu.sync_copy(idx_hbm.at[pl.ds(row * KPAD, KPAD)], idx_v)
        pltpu.sync_copy(w_hbm.at[pl.ds(row * KPAD * 16, KPAD * 16)], w_v)
        pltpu.async_copy(emb_hbm.at[idx_v], rows_v, sem).wait()
        for c in range(D // 16):
            acc_v[pl.ds(c * 16, 16)] = jnp.zeros((16,), jnp.float32)

        def body(j, carry):
            wb = w_v[pl.ds(j * 16, 16)]
            for c in range(D // 16):
                sl = pl.ds(c * 16, 16)
                acc_v[sl] = acc_v[sl] + wb * rows_v[j, sl]
            return carry

        lax.fori_loop(0, KPAD, body, 0)
        pltpu.sync_copy(acc_v, out_hbm.at[row])


_MIX_CACHE = []


def _mix(emb_weight, idx, w):
    if not _MIX_CACHE:
        _MIX_CACHE.append(functools.partial(
            pl.kernel,
            mesh=plsc.VectorSubcoreMesh(core_axis_name="c", subcore_axis_name="s"),
            out_type=jax.ShapeDtypeStruct((B, D), jnp.float32),
            scratch_types=[
                pltpu.VMEM((KPAD,), jnp.int32),
                pltpu.VMEM((KPAD * 16,), jnp.float32),
                pltpu.VMEM((KPAD, D), jnp.float32),
                pltpu.VMEM((D,), jnp.float32),
                pltpu.SemaphoreType.DMA,
            ],
        )(_mix_body))
    return _MIX_CACHE[0](emb_weight, idx, w)


def kernel(logits, emb_weight):
    assert logits.shape == (B, 128000) and emb_weight.shape == (128000, D)
    w, idx = _topk(logits)
    return _mix(emb_weight, idx, w)
